# baseline (speedup 1.0000x reference)
import functools
import math

import jax
import jax.numpy as jnp
from jax import lax
from jax.experimental import pallas as pl
from jax.experimental.pallas import tpu as pltpu
from jax.experimental.pallas import tpu_sc as plsc

C = 128
NG = 64
N0, N1, N2 = 50000, 50000, 15000
NSLOT = 6
ROW = NSLOT * NG
L = 16
NC, NS = 2, 16
NW = NC * NS
CB = 2048

SWW01, AWW01 = 25000, 784
SWW2, AWW2 = 7504, 240
BW01, BWW01 = 12504, 400
BW2, BWW2 = 3752, 128

_f32 = jnp.float32
_i32 = jnp.int32

_SC_PARAMS = pltpu.CompilerParams(needs_layout_passes=False)
_MESH = plsc.VectorSubcoreMesh(core_axis_name="c", subcore_axis_name="s",
                               num_cores=NC, num_subcores=NS)


def _wid():
  return lax.axis_index("s") * NC + lax.axis_index("c")


def _clamp8(raw, total, w):
  return pl.multiple_of(jnp.minimum(raw, total - w), 8)


def _stage_a_body(x0r, x1r, x2r, b0r, b1r, b2r, wsumr,
                  t_up0, t_bd1, t_d1, t_u1, t_bd2, t_d2,
                  o_b0, o_b1, o_b2,
                  wtab, xrows, pswa, pswb, pswc, bwin, pbat, sem):
  wid = _wid()
  lanes = lax.iota(_i32, L)
  pltpu.sync_copy(wsumr, wtab)

  def job_x(xr, tot, ww, tables):
    wbase = _clamp8(wid * ww, tot, ww)
    pltpu.sync_copy(xr.at[pl.ds(wbase * 8, 8 * ww)],
                    xrows.at[pl.ds(0, 8 * ww)])
    for t in range(ww // L):
      wloc = lanes + t * L
      fe = wloc * 8
      xe = [plsc.load_gather(xrows, [fe + j]) for j in range(4)]
      xo = [plsc.load_gather(xrows, [fe + (4 + j)]) for j in range(4)]
      for (_, w_slot, buf) in tables:
        wsl = jnp.full((L,), w_slot, _i32)
        ve = plsc.load_gather(wtab, [wsl, xe[0]])
        vo = plsc.load_gather(wtab, [wsl, xo[0]])
        for j in range(1, 4):
          ve = ve + plsc.load_gather(wtab, [wsl, xe[j]])
          vo = vo + plsc.load_gather(wtab, [wsl, xo[j]])
        be = plsc.bitcast(ve, _i32) + 32768
        bo = plsc.bitcast(vo, _i32) + 32768
        word = lax.shift_right_logical(be, 16) | (bo & jnp.int32(-65536))
        buf[pl.ds(t * L, L)] = word
    for (out_ref, _, buf) in tables:
      pltpu.sync_copy(buf.at[pl.ds(0, ww)], out_ref.at[pl.ds(wbase, ww)])

  def job_b(br, tot, ww, out_ref):
    wbase = _clamp8(wid * ww, tot, ww)
    pltpu.sync_copy(br.at[pl.ds(wbase * 4, 4 * ww)],
                    bwin.at[pl.ds(0, 4 * ww)])
    for t in range(ww // L):
      wloc = lanes + t * L
      p = wloc * 4
      word = plsc.load_gather(bwin, [p])
      for k in range(1, 4):
        word = word | lax.shift_left(plsc.load_gather(bwin, [p + k]), 8 * k)
      pbat[pl.ds(t * L, L)] = word
    pltpu.sync_copy(pbat.at[pl.ds(0, ww)], out_ref.at[pl.ds(wbase, ww)])

  job_x(x0r, SWW01, AWW01, [(t_up0, 0, pswa)])
  job_x(x1r, SWW01, AWW01,
        [(t_bd1, 1, pswa), (t_d1, 2, pswb), (t_u1, 3, pswc)])
  job_x(x2r, SWW2, AWW2, [(t_bd2, 5, pswa), (t_d2, 4, pswb)])
  job_b(b0r, BW01, BWW01, o_b0)
  job_b(b1r, BW01, BWW01, o_b1)
  job_b(b2r, BW2, BWW2, o_b2)


_stage_a = functools.partial(
    pl.kernel,
    out_type=(
        jax.ShapeDtypeStruct((SWW01,), _i32),
        jax.ShapeDtypeStruct((SWW01,), _i32),
        jax.ShapeDtypeStruct((SWW01,), _i32),
        jax.ShapeDtypeStruct((SWW01,), _i32),
        jax.ShapeDtypeStruct((SWW2,), _i32),
        jax.ShapeDtypeStruct((SWW2,), _i32),
        jax.ShapeDtypeStruct((BW01,), _i32),
        jax.ShapeDtypeStruct((BW01,), _i32),
        jax.ShapeDtypeStruct((BW2,), _i32),
    ),
    mesh=_MESH,
    compiler_params=_SC_PARAMS,
    scratch_types=[
        pltpu.VMEM((NSLOT, C), _f32),
        pltpu.VMEM((8 * AWW01,), _i32),
        pltpu.VMEM((AWW01,), _i32),
        pltpu.VMEM((AWW01,), _i32),
        pltpu.VMEM((AWW01,), _i32),
        pltpu.VMEM((4 * BWW01,), _i32),
        pltpu.VMEM((BWW01,), _i32),
        pltpu.SemaphoreType.DMA,
    ],
)(_stage_a_body)


def _cpw(E):
  return math.ceil(math.ceil(E / CB) / NW)


def _stage_b_body(s_up0, s_bd1, s_d1, s_u1, s_bd2, s_d2,
                  pb0, pb1, pb2, rb0, rb1, rb2,
                  up0v, up0g, bd1v, bd1g, d1v, d1g, u1v, u1g,
                  bd2v, bd2g, d2v, d2g, zr, outr,
                  acc, vidx0, vidx1, gidx0, gidx1, swp, batp,
                  semLv, semLg, semSW, semBT):
  wid = _wid()
  lanes = lax.iota(_i32, L)
  lane_rows = lanes * ROW
  ones = jnp.ones((L,), _f32)
  vidxs = (vidx0, vidx1)
  gidxs = (gidx0, gidx1)

  pltpu.sync_copy(zr, acc)

  def chunk_m0_base(ci, E):
    m0 = jnp.maximum(0, ci * CB - (E - CB))
    return m0, pl.multiple_of(ci * CB - m0, 8)

  def do_edges(ev, eg, off, E):
    cpw = _cpw(E)
    start = wid * cpw
    m0s, Ls = {}, {}

    def issue_l(k):
      m0, base = chunk_m0_base(start + k, E)
      m0s[k] = m0
      b = k % 2
      Ls[k] = (
          pltpu.async_copy(ev.at[pl.ds(base, CB)], vidxs[b], semLv),
          pltpu.async_copy(eg.at[pl.ds(base, CB)], gidxs[b], semLg),
      )

    issue_l(0)
    for k in range(cpw):
      for cp in Ls[k]:
        cp.wait()
      if k + 1 < cpw:
        issue_l(k + 1)
      b = k % 2
      m0 = m0s[k]

      def tbody(t, carry):
        pos = lanes + t * L
        vi = plsc.load_gather(vidxs[b], [pos])
        gi = plsc.load_gather(gidxs[b], [pos])
        w32 = plsc.load_gather(swp, [lax.shift_right_logical(vi, 1)])
        bits = jnp.where((vi & 1) == 1, w32 & jnp.int32(-65536),
                         lax.shift_left(w32, 16))
        val = plsc.bitcast(bits, _f32)
        bw = plsc.load_gather(batp, [lax.shift_right_logical(gi, 2)])
        g = lax.shift_right_logical(bw, lax.shift_left(gi & 3, 3)) & 255
        mask = pos >= m0
        plsc.addupdate_scatter(acc, [lane_rows + (off + g)], val, mask=mask)
        return carry

      lax.fori_loop(0, CB // L, tbody, 0)

  def do_counts(batr, off, N):
    cpw = _cpw(N)
    start = wid * cpw
    m0s, Ls = {}, {}

    def issue_l(k):
      m0, base = chunk_m0_base(start + k, N)
      m0s[k] = m0
      Ls[k] = pltpu.async_copy(batr.at[pl.ds(base, CB)], vidxs[k % 2], semLv)

    issue_l(0)
    for k in range(cpw):
      Ls[k].wait()
      if k + 1 < cpw:
        issue_l(k + 1)
      b = k % 2
      m0 = m0s[k]

      def tbody(t, carry):
        pos = lanes + t * L
        g = plsc.load_gather(vidxs[b], [pos])
        mask = pos >= m0
        plsc.addupdate_scatter(acc, [lane_rows + (off + g)], ones, mask=mask)
        return carry

      lax.fori_loop(0, CB // L, tbody, 0)

  def load_sw(src, ln):
    return pltpu.async_copy(src, swp.at[pl.ds(0, ln)], semSW)

  def load_bat(src, ln):
    return pltpu.async_copy(src, batp.at[pl.ds(0, ln)], semBT)

  cpb = load_bat(pb0, BW01)
  cps = load_sw(s_up0, SWW01)
  cpb.wait()
  cps.wait()
  do_edges(up0v, up0g, 0 * NG, 100000)
  load_sw(s_bd1, SWW01).wait()
  do_edges(bd1v, bd1g, 0 * NG, 100000)
  do_counts(rb0, 3 * NG, N0)
  cpb = load_bat(pb1, BW01)
  cps = load_sw(s_d1, SWW01)
  cpb.wait()
  cps.wait()
  do_edges(d1v, d1g, 1 * NG, 200000)
  load_sw(s_u1, SWW01).wait()
  do_edges(u1v, u1g, 1 * NG, 90000)
  load_sw(s_bd2, SWW2).wait()
  do_edges(bd2v, bd2g, 1 * NG, 45000)
  do_counts(rb1, 4 * NG, N1)
  cpb = load_bat(pb2, BW2)
  cps = load_sw(s_d2, SWW2)
  cpb.wait()
  cps.wait()
  do_edges(d2v, d2g, 2 * NG, 60000)
  do_counts(rb2, 5 * NG, N2)

  pltpu.sync_copy(acc, outr.at[wid])


_stage_b = functools.partial(
    pl.kernel,
    out_type=jax.ShapeDtypeStruct((NW, L * ROW), _f32),
    mesh=_MESH,
    compiler_params=_SC_PARAMS,
    scratch_types=[
        pltpu.VMEM((L * ROW,), _f32),
        pltpu.VMEM((CB,), _i32),
        pltpu.VMEM((CB,), _i32),
        pltpu.VMEM((CB,), _i32),
        pltpu.VMEM((CB,), _i32),
        pltpu.VMEM((SWW01,), _i32),
        pltpu.VMEM((BW01,), _i32),
        pltpu.SemaphoreType.DMA,
        pltpu.SemaphoreType.DMA,
        pltpu.SemaphoreType.DMA,
        pltpu.SemaphoreType.DMA,
    ],
)(_stage_b_body)


def _wsum_body(w_ref, o_ref):
  o_ref[...] = jnp.sum(w_ref[...], axis=2)


def _combine_body(t0, t1, t2, n0, n1, n2, b_ref, o_ref):
  T0 = jnp.sum(t0[...], axis=0)
  T1 = jnp.sum(t1[...], axis=0)
  T2 = jnp.sum(t2[...], axis=0)
  c0v = jnp.sum(n0[...], axis=0)
  c1v = jnp.sum(n1[...], axis=0)
  c2v = jnp.sum(n2[...], axis=0)
  b0s = jnp.sum(b_ref[0:2, :])
  b1s = jnp.sum(b_ref[2:4, :]) + jnp.sum(b_ref[5:6, :])
  b2s = jnp.sum(b_ref[4:5, :])
  tot = (T0 + c0v * b0s) / jnp.maximum(c0v, 1.0)
  tot = tot + (T1 + c1v * b1s) / jnp.maximum(c1v, 1.0)
  tot = tot + (T2 + c2v * b2s) / jnp.maximum(c2v, 1.0)
  o_ref[...] = jax.nn.sigmoid(tot)[None, :]


def kernel(x0, x1, x2, up_index_0, boundary_index_1, down_index_1, up_index_1,
           boundary_index_2, down_index_2, batch0, batch1, batch2,
           W0, b0, W1, b1, W2, b2, W3, b3, W4, b4, W5, b5):
  wstack = jnp.stack([W0, W1, W2, W3, W4, W5]).astype(_f32)
  bstack = jnp.stack([b0, b1, b2, b3, b4, b5]).astype(_f32)

  wsum = pl.pallas_call(
      _wsum_body,
      out_shape=jax.ShapeDtypeStruct((NSLOT, C), _f32),
  )(wstack)

  def r(a):
    return a.astype(_i32)

  x2p = jnp.pad(r(x2), ((0, 8), (0, 0))).reshape(-1)
  b0p = jnp.pad(r(batch0), (0, 16))
  b1p = jnp.pad(r(batch1), (0, 16))
  b2p = jnp.pad(r(batch2), (0, 8))

  packed = _stage_a(r(x0).reshape(-1), r(x1).reshape(-1), x2p,
                    b0p, b1p, b2p, wsum)

  partials = _stage_b(
      *packed,
      b0p, b1p, b2p,
      r(up_index_0[0]), r(up_index_0[1]),
      r(boundary_index_1[1]), r(boundary_index_1[0]),
      r(down_index_1[0]), r(down_index_1[1]),
      r(up_index_1[0]), r(up_index_1[1]),
      r(boundary_index_2[1]), r(boundary_index_2[0]),
      r(down_index_2[0]), r(down_index_2[1]),
      jnp.zeros((L * ROW,), _f32))

  p = partials.reshape(NW * L, NSLOT, NG)
  out = pl.pallas_call(
      _combine_body,
      out_shape=jax.ShapeDtypeStruct((1, NG), _f32),
  )(p[:, 0, :], p[:, 1, :], p[:, 2, :], p[:, 3, :], p[:, 4, :], p[:, 5, :],
    bstack)
  return out.reshape(NG)


# speedup vs baseline: 1.0379x; 1.0379x over previous
import functools
import math

import jax
import jax.numpy as jnp
from jax import lax
from jax.experimental import pallas as pl
from jax.experimental.pallas import tpu as pltpu
from jax.experimental.pallas import tpu_sc as plsc

C = 128
NG = 64
N0, N1, N2 = 50000, 50000, 15000
NSLOT = 6
ROW = NSLOT * NG
L = 16
NC, NS = 2, 16
NW = NC * NS
CB = 2048

_f32 = jnp.float32
_i32 = jnp.int32

_SC_PARAMS = pltpu.CompilerParams(needs_layout_passes=False)
_MESH = plsc.VectorSubcoreMesh(core_axis_name="c", subcore_axis_name="s",
                               num_cores=NC, num_subcores=NS)


def _wwin(total):
  return math.ceil(total / (NW * L)) * L


_WA = _wwin(N0)


def _wid():
  return lax.axis_index("s") * NC + lax.axis_index("c")


def _clamp8(raw, total, w):
  return pl.multiple_of(jnp.minimum(raw, total - w), 8)


def _stage_a_body(x0r, x1r, x2r, wsumr,
                  t_up0, t_bd1, t_d1, t_u1, t_bd2, t_d2,
                  wtab, xrows, bufa, bufb, bufc, sem):
  wid = _wid()
  lanes = lax.iota(_i32, L)
  pltpu.sync_copy(wsumr, wtab)

  def job_x(xr, N, tables):
    ww = _wwin(N)
    wbase = _clamp8(wid * ww, N, ww)
    pltpu.sync_copy(xr.at[pl.ds(wbase * 4, 4 * ww)],
                    xrows.at[pl.ds(0, 4 * ww)])

    for t in range(ww // L):
      pos = lanes + t * L
      f = pos * 4
      xv = [plsc.load_gather(xrows, [f + j]) for j in range(4)]
      for (_, w_slot, buf) in tables:
        wsl = jnp.full((L,), w_slot, _i32)
        val = plsc.load_gather(wtab, [wsl, xv[0]])
        for j in range(1, 4):
          val = val + plsc.load_gather(wtab, [wsl, xv[j]])
        buf[pl.ds(t * L, L)] = val
    for (out_ref, _, buf) in tables:
      pltpu.sync_copy(buf.at[pl.ds(0, ww)], out_ref.at[pl.ds(wbase, ww)])

  job_x(x0r, N0, [(t_up0, 0, bufa)])
  job_x(x1r, N1, [(t_bd1, 1, bufa), (t_d1, 2, bufb), (t_u1, 3, bufc)])
  job_x(x2r, N2, [(t_bd2, 5, bufa), (t_d2, 4, bufb)])


_stage_a = functools.partial(
    pl.kernel,
    out_type=(
        jax.ShapeDtypeStruct((N0,), _f32),
        jax.ShapeDtypeStruct((N1,), _f32),
        jax.ShapeDtypeStruct((N1,), _f32),
        jax.ShapeDtypeStruct((N1,), _f32),
        jax.ShapeDtypeStruct((N2,), _f32),
        jax.ShapeDtypeStruct((N2,), _f32),
    ),
    mesh=_MESH,
    compiler_params=_SC_PARAMS,
    scratch_types=[
        pltpu.VMEM((NSLOT, C), _f32),
        pltpu.VMEM((4 * _WA,), _i32),
        pltpu.VMEM((_WA,), _f32),
        pltpu.VMEM((_WA,), _f32),
        pltpu.VMEM((_WA,), _f32),
        pltpu.SemaphoreType.DMA,
    ],
)(_stage_a_body)


def _cpw(E):
  return math.ceil(math.ceil(E / CB) / NW)


def _stage_b_body(s_up0, s_bd1, s_d1, s_u1, s_bd2, s_d2,
                  rb0, rb1, rb2,
                  up0v, up0g, bd1v, bd1g, d1v, d1g, u1v, u1g,
                  bd2v, bd2g, d2v, d2g, zr, outr,
                  acc, vidx0, vidx1, gidx0, gidx1, swt, btab,
                  semLv, semLg, semSW, semBT):
  wid = _wid()
  lanes = lax.iota(_i32, L)
  lane_rows = lanes * ROW
  ones = jnp.ones((L,), _f32)
  vidxs = (vidx0, vidx1)
  gidxs = (gidx0, gidx1)

  pltpu.sync_copy(zr, acc)

  def chunk_m0_base(ci, E):
    m0 = jnp.maximum(0, ci * CB - (E - CB))
    return m0, pl.multiple_of(ci * CB - m0, 8)

  def do_edges(ev, eg, off, E):
    cpw = _cpw(E)
    start = wid * cpw
    m0s, Ls = {}, {}

    def issue_l(k):
      m0, base = chunk_m0_base(start + k, E)
      m0s[k] = m0
      b = k % 2
      Ls[k] = (
          pltpu.async_copy(ev.at[pl.ds(base, CB)], vidxs[b], semLv),
          pltpu.async_copy(eg.at[pl.ds(base, CB)], gidxs[b], semLg),
      )

    issue_l(0)
    for k in range(cpw):
      for cp in Ls[k]:
        cp.wait()
      if k + 1 < cpw:
        issue_l(k + 1)
      b = k % 2
      m0 = m0s[k]

      def tbody(t, carry):
        pos = lanes + t * L
        vi = plsc.load_gather(vidxs[b], [pos])
        gi = plsc.load_gather(gidxs[b], [pos])
        val = plsc.load_gather(swt, [vi])
        g = plsc.load_gather(btab, [gi])
        mask = pos >= m0
        plsc.addupdate_scatter(acc, [lane_rows + (off + g)], val, mask=mask)
        return carry

      lax.fori_loop(0, CB // L, tbody, 0)

  def do_counts(off, N):
    cpn = math.ceil(N / NW)
    a0 = wid * cpn
    a1 = jnp.minimum(N, a0 + cpn)

    def tbody(t, carry):
      idx = a0 + t * L + lanes
      mask = idx < a1
      g = plsc.load_gather(btab, [jnp.minimum(idx, N - 1)])
      plsc.addupdate_scatter(acc, [lane_rows + (off + g)], ones, mask=mask)
      return carry

    lax.fori_loop(0, math.ceil(cpn / L), tbody, 0)

  def load_sw(src, ln):
    return pltpu.async_copy(src, swt.at[pl.ds(0, ln)], semSW)

  def load_bat(src, ln):
    return pltpu.async_copy(src, btab.at[pl.ds(0, ln)], semBT)

  cpb = load_bat(rb0, N0)
  cps = load_sw(s_up0, N0)
  cpb.wait()
  cps.wait()
  do_edges(up0v, up0g, 0 * NG, 100000)
  load_sw(s_bd1, N1).wait()
  do_edges(bd1v, bd1g, 0 * NG, 100000)
  do_counts(3 * NG, N0)
  cpb = load_bat(rb1, N1)
  cps = load_sw(s_d1, N1)
  cpb.wait()
  cps.wait()
  do_edges(d1v, d1g, 1 * NG, 200000)
  load_sw(s_u1, N1).wait()
  do_edges(u1v, u1g, 1 * NG, 90000)
  load_sw(s_bd2, N2).wait()
  do_edges(bd2v, bd2g, 1 * NG, 45000)
  do_counts(4 * NG, N1)
  cpb = load_bat(rb2, N2)
  cps = load_sw(s_d2, N2)
  cpb.wait()
  cps.wait()
  do_edges(d2v, d2g, 2 * NG, 60000)
  do_counts(5 * NG, N2)

  pltpu.sync_copy(acc, outr.at[wid])


_stage_b = functools.partial(
    pl.kernel,
    out_type=jax.ShapeDtypeStruct((NW, L * ROW), _f32),
    mesh=_MESH,
    compiler_params=_SC_PARAMS,
    scratch_types=[
        pltpu.VMEM((L * ROW,), _f32),
        pltpu.VMEM((CB,), _i32),
        pltpu.VMEM((CB,), _i32),
        pltpu.VMEM((CB,), _i32),
        pltpu.VMEM((CB,), _i32),
        pltpu.VMEM((N0,), _f32),
        pltpu.VMEM((N0,), _i32),
        pltpu.SemaphoreType.DMA,
        pltpu.SemaphoreType.DMA,
        pltpu.SemaphoreType.DMA,
        pltpu.SemaphoreType.DMA,
    ],
)(_stage_b_body)


def _wsum_body(w_ref, o_ref):
  o_ref[...] = jnp.sum(w_ref[...], axis=2)


def _combine_body(t0, t1, t2, n0, n1, n2, b_ref, o_ref):
  T0 = jnp.sum(t0[...], axis=0)
  T1 = jnp.sum(t1[...], axis=0)
  T2 = jnp.sum(t2[...], axis=0)
  c0v = jnp.sum(n0[...], axis=0)
  c1v = jnp.sum(n1[...], axis=0)
  c2v = jnp.sum(n2[...], axis=0)
  b0s = jnp.sum(b_ref[0:2, :])
  b1s = jnp.sum(b_ref[2:4, :]) + jnp.sum(b_ref[5:6, :])
  b2s = jnp.sum(b_ref[4:5, :])
  tot = (T0 + c0v * b0s) / jnp.maximum(c0v, 1.0)
  tot = tot + (T1 + c1v * b1s) / jnp.maximum(c1v, 1.0)
  tot = tot + (T2 + c2v * b2s) / jnp.maximum(c2v, 1.0)
  o_ref[...] = jax.nn.sigmoid(tot)[None, :]


def kernel(x0, x1, x2, up_index_0, boundary_index_1, down_index_1, up_index_1,
           boundary_index_2, down_index_2, batch0, batch1, batch2,
           W0, b0, W1, b1, W2, b2, W3, b3, W4, b4, W5, b5):
  wstack = jnp.stack([W0, W1, W2, W3, W4, W5]).astype(_f32)
  bstack = jnp.stack([b0, b1, b2, b3, b4, b5]).astype(_f32)

  wsum = pl.pallas_call(
      _wsum_body,
      out_shape=jax.ShapeDtypeStruct((NSLOT, C), _f32),
  )(wstack)

  def r(a):
    return a.astype(_i32)

  sw = _stage_a(r(x0).reshape(-1), r(x1).reshape(-1), r(x2).reshape(-1),
                wsum)

  partials = _stage_b(
      *sw,
      r(batch0), r(batch1), r(batch2),
      r(up_index_0[0]), r(up_index_0[1]),
      r(boundary_index_1[1]), r(boundary_index_1[0]),
      r(down_index_1[0]), r(down_index_1[1]),
      r(up_index_1[0]), r(up_index_1[1]),
      r(boundary_index_2[1]), r(boundary_index_2[0]),
      r(down_index_2[0]), r(down_index_2[1]),
      jnp.zeros((L * ROW,), _f32))

  p = partials.reshape(NW * L, NSLOT, NG)
  out = pl.pallas_call(
      _combine_body,
      out_shape=jax.ShapeDtypeStruct((1, NG), _f32),
  )(p[:, 0, :], p[:, 1, :], p[:, 2, :], p[:, 3, :], p[:, 4, :], p[:, 5, :],
    bstack)
  return out.reshape(NG)
